# baseline (speedup 1.0000x reference)
import jax
import jax.numpy as jnp
from jax import lax
from jax.experimental import pallas as pl
from jax.experimental.pallas import tpu as pltpu
from jax.experimental.pallas import tpu_sc as plsc

N = 10000
E = 160000
D = 128
DO = 127

NT = 32
CPT = 320
WPC = 313
BMW = 100160
CHUNK = 1600
NCH = E // CHUNK
PCAP = 3232
OCAP = 4096
FLUSH = 512
ROWCAP = 321536
CNTP = 384
ACCR = 336

_i32 = jnp.int32


def _iota16():
  return lax.iota(_i32, 16)


def _splat(x):
  return jnp.zeros((16,), _i32) + x


def _append_ring(ref, ocur, vals, mask):
  rank = plsc.cumsum(jnp.ones((16,), _i32), mask=mask)
  pos = (ocur + rank - 1) & (OCAP - 1)
  plsc.store_scatter(ref, [pos], vals, mask=mask)
  return ocur + plsc.all_reduce_population_count(mask)


def _k1_body(col_hbm, src_hbm, dst_hbm, pairs_hbm, lens_hbm, cnt_hbm,
             col_v, bm, pend, ostage, cnt_v, sbuf, dbuf, sbuf2, dbuf2, lenv,
             sem_as, sem_ad, sem_bs, sem_bd):
  core = lax.axis_index("c")
  sub = lax.axis_index("s")
  wid = sub * 2 + core
  lo = wid * CPT
  hi = jnp.minimum(lo + CPT, N)
  iot = _iota16()

  pltpu.sync_copy(col_hbm, col_v)

  def _zero_bm(i, _):
    bm[pl.ds(i * 16, 16)] = jnp.zeros((16,), _i32)
    return 0
  lax.fori_loop(0, BMW // 16, _zero_bm, 0)

  def _zero_cnt(i, _):
    cnt_v[pl.ds(i * 16, 16)] = jnp.zeros((16,), _i32)
    return 0
  lax.fori_loop(0, CNTP // 16, _zero_cnt, 0)

  def _self(i, ocur):
    k = i * 16 + iot
    m = k < (hi - lo)
    kc = jnp.where(m, k, 0)
    c = lo + kc
    word = kc * WPC + (c >> 5)
    bit = jnp.left_shift(_i32(1), c & 31)
    plsc.store_scatter(bm, [word], bit, mask=m)
    vn = plsc.load_gather(col_v, [c], mask=m)
    keep = m & (vn == 0.0)
    plsc.addupdate_scatter(cnt_v, [kc], jnp.ones((16,), _i32), mask=keep)
    return _append_ring(ostage, ocur, (c << 14) | c, mask=keep)
  ocur = lax.fori_loop(0, (CPT + 15) // 16, _self, jnp.zeros((16,), _i32))

  def _consume(pcur, ocur):
    def body(i, oc):
      base = i * 16
      lv = (base + iot) < pcur
      p = pend[pl.ds(base, 16)]
      c = jnp.where(lv, p >> 14, lo)
      n = jnp.where(lv, p & 16383, 0)
      k = c - lo
      word = k * WPC + (n >> 5)
      bit = jnp.left_shift(_i32(1), n & 31)
      rc, _ = plsc.scan_count(p, mask=lv)
      first = lv & (rc == 1)
      w = plsc.load_gather(bm, [word], mask=lv)
      fresh = first & ((w & bit) == 0)
      def wave_cond(a):
        return jnp.max(a) > 0
      def wave_body(a):
        ab = a > 0
        rcw, _ = plsc.scan_count(word, mask=ab)
        lead = ab & (rcw == 1)
        plsc.addupdate_scatter(bm, [word], bit, mask=lead)
        return jnp.where(lead, 0, a)
      lax.while_loop(wave_cond, wave_body, jnp.where(fresh, 1, 0))
      vn = plsc.load_gather(col_v, [n], mask=lv)
      keep = fresh & (vn == 0.0)
      rc2, last2 = plsc.scan_count(k, mask=keep)
      plsc.addupdate_scatter(cnt_v, [k], rc2, mask=keep & last2)
      return _append_ring(ostage, oc, p, mask=keep)
    return lax.fori_loop(0, (pcur + 15) // 16, body, ocur)

  def _process(sbuf, dbuf, carry):
    ocur, flushed = carry

    nc = hi - lo
    @plsc.parallel_loop(0, CHUNK // 16, unroll=8,
                        carry=jnp.zeros((16,), _i32))
    def scan(j, pcur):
      s = sbuf[pl.ds(j * 16, 16)]
      d = dbuf[pl.ds(j * 16, 16)]
      for c, n in ((s, d), (d, s)):
        m = (c - lo).astype(jnp.uint32) < nc.astype(jnp.uint32)
        pk = (c << 14) | n
        rank = plsc.cumsum(jnp.ones((16,), _i32), mask=m)
        plsc.store_scatter(pend, [pcur + rank - 1], pk, mask=m)
        pcur = pcur + plsc.all_reduce_population_count(m)
      return pcur
    ocur = _consume(jnp.max(scan), ocur)

    def fl_cond(st):
      oc, fl = st
      return (oc - fl) >= FLUSH
    def fl_body(st):
      oc, fl = st
      pltpu.sync_copy(ostage.at[pl.ds(pl.multiple_of(fl & (OCAP - 1), 512), FLUSH)],
                      pairs_hbm.at[pl.ds(pl.multiple_of(wid * ROWCAP + fl, 512), FLUSH)])
      return oc, fl + FLUSH
    ocur_s, flushed = lax.while_loop(fl_cond, fl_body, (jnp.max(ocur), flushed))
    return ocur, flushed

  def _start(ch, sb, db, ss, sd):
    off = pl.multiple_of(ch * CHUNK, 8)
    pltpu.async_copy(src_hbm.at[pl.ds(off, CHUNK)], sb, ss)
    pltpu.async_copy(dst_hbm.at[pl.ds(off, CHUNK)], db, sd)

  def _waitc(ch, sb, db, ss, sd):
    off = pl.multiple_of(ch * CHUNK, 8)
    pltpu.make_async_copy(src_hbm.at[pl.ds(off, CHUNK)], sb, ss).wait()
    pltpu.make_async_copy(dst_hbm.at[pl.ds(off, CHUNK)], db, sd).wait()

  _start(0, sbuf, dbuf, sem_as, sem_ad)

  def _ring(i, carry):
    cha = 2 * i
    _waitc(cha, sbuf, dbuf, sem_as, sem_ad)
    _start(cha + 1, sbuf2, dbuf2, sem_bs, sem_bd)
    carry = _process(sbuf, dbuf, carry)
    _waitc(cha + 1, sbuf2, dbuf2, sem_bs, sem_bd)
    _start(jnp.minimum(cha + 2, NCH - 2), sbuf, dbuf, sem_as, sem_ad)
    carry = _process(sbuf2, dbuf2, carry)
    return carry

  ocur, flushed = lax.fori_loop(0, NCH // 2, _ring, (ocur, _i32(0)))
  _waitc(NCH - 2, sbuf, dbuf, sem_as, sem_ad)

  pltpu.sync_copy(ostage.at[pl.ds(pl.multiple_of(flushed & (OCAP - 1), 512), FLUSH)],
                  pairs_hbm.at[pl.ds(pl.multiple_of(wid * ROWCAP + flushed, 512), FLUSH)])

  for j in range(8):
    lenv[pl.ds(j * 16, 16)] = ocur
  pltpu.sync_copy(lenv, lens_hbm.at[pl.ds(pl.multiple_of(wid * 128, 128), 128)])
  pltpu.sync_copy(cnt_v, cnt_hbm.at[pl.ds(pl.multiple_of(wid * CNTP, 128), CNTP)])


def _k2_body(x_hbm, zeros_hbm, pairs_hbm, lens_hbm, cnt_hbm, out_hbm,
             stage, pendb, accall, outall, cntv, lenv, acc_sh):
  core = lax.axis_index("c")
  sub = lax.axis_index("s")
  wid = sub * 2 + core
  lo = wid * CPT
  hi = jnp.minimum(lo + CPT, N)
  cw = hi - lo
  iot = _iota16()
  arow = sub * ACCR

  pltpu.sync_copy(zeros_hbm, acc_sh.at[pl.ds(pl.multiple_of(arow, 16), ACCR)])

  pltpu.sync_copy(lens_hbm.at[pl.ds(pl.multiple_of(wid * 128, 128), 128)], lenv)
  ln = jnp.max(lenv[pl.ds(0, 16)])
  pltpu.sync_copy(cnt_hbm.at[pl.ds(pl.multiple_of(wid * CNTP, 128), CNTP)], cntv)

  def ch_cond(ch):
    return ch * 2048 < ln
  def ch_body(ch):
    pltpu.sync_copy(pairs_hbm.at[pl.ds(pl.multiple_of(wid * ROWCAP + ch * 2048, 512), 2048)], pendb)
    def batch(b, _):
      g0 = ch * 2048 + b * 16
      gm = (g0 + iot) < ln
      p = pendb[pl.ds(b * 16, 16)]
      n = jnp.where(gm, p & 16383, 0)
      k = jnp.where(gm, (p >> 14) - lo, CPT)
      pltpu.sync_copy(x_hbm.at[n], stage)
      pltpu.sync_copy(stage, acc_sh.at[arow + k], add=True)
      return 0
    lax.fori_loop(0, 128, batch, 0)
    return ch + 1
  lax.while_loop(ch_cond, ch_body, _i32(0))

  pltpu.sync_copy(acc_sh.at[pl.ds(pl.multiple_of(arow, 16), CPT)], accall)

  @plsc.parallel_loop(0, CPT, unroll=2)
  def _row(r):
    cs = plsc.load_gather(cntv, [_splat(r)])
    cf = cs.astype(jnp.float32)
    for j in range(D // 16):
      v = accall[r, pl.ds(j * 16, 16)]
      col = j * 16 + iot
      plsc.store_scatter(outall, [_splat(r), col], v / cf,
                         mask=col < DO)

  for t in range(CPT // 80):
    row0 = jnp.minimum(t * 80, cw - 80)
    pltpu.sync_copy(outall.at[pl.ds(pl.multiple_of(row0, 8), 80)],
                    out_hbm.at[pl.ds(pl.multiple_of(lo + row0, 8), 80)])


def _mesh():
  return plsc.VectorSubcoreMesh(core_axis_name="c", subcore_axis_name="s")


_CP = pltpu.CompilerParams(use_tc_tiling_on_sc=False,
                           needs_layout_passes=False)


@jax.jit
def kernel(x, edge_index):
  col = x[:, D - 1]
  k1 = pl.kernel(
      _k1_body,
      out_type=(
          jax.ShapeDtypeStruct((NT * ROWCAP,), _i32),
          jax.ShapeDtypeStruct((NT * 128,), _i32),
          jax.ShapeDtypeStruct((NT * CNTP,), _i32),
      ),
      mesh=_mesh(),
      compiler_params=_CP,
      scratch_types=[
          pltpu.VMEM((N,), jnp.float32),
          pltpu.VMEM((BMW,), _i32),
          pltpu.VMEM((PCAP,), _i32),
          pltpu.VMEM((OCAP,), _i32),
          pltpu.VMEM((CNTP,), _i32),
          pltpu.VMEM((CHUNK,), _i32),
          pltpu.VMEM((CHUNK,), _i32),
          pltpu.VMEM((CHUNK,), _i32),
          pltpu.VMEM((CHUNK,), _i32),
          pltpu.VMEM((128,), _i32),
          pltpu.SemaphoreType.DMA,
          pltpu.SemaphoreType.DMA,
          pltpu.SemaphoreType.DMA,
          pltpu.SemaphoreType.DMA,
      ],
  )
  pairs, lens, cnt = k1(col, edge_index[0], edge_index[1])

  k2 = pl.kernel(
      _k2_body,
      out_type=jax.ShapeDtypeStruct((N, DO), jnp.float32),
      mesh=_mesh(),
      compiler_params=_CP,
      scratch_types=[
          pltpu.VMEM((16, D), jnp.float32),
          pltpu.VMEM((2048,), _i32),
          pltpu.VMEM((CPT, D), jnp.float32),
          pltpu.VMEM((CPT, DO), jnp.float32),
          pltpu.VMEM((CNTP,), _i32),
          pltpu.VMEM((128,), _i32),
          pltpu.VMEM_SHARED((16 * ACCR, D), jnp.float32),
      ],
  )
  zeros = jnp.zeros((ACCR, D), jnp.float32)
  return k2(x, zeros, pairs, lens, cnt)


# speedup vs baseline: 1.1971x; 1.1971x over previous
import jax
import jax.numpy as jnp
from jax import lax
from jax.experimental import pallas as pl
from jax.experimental.pallas import tpu as pltpu
from jax.experimental.pallas import tpu_sc as plsc

N = 10000
E = 160000
D = 128
DO = 127

NT = 32
CPT = 320
WPC = 313
BMW = 100160
CHUNK = 1600
NCH = E // CHUNK
PCAP = 3232
OCAP = 4096
FLUSH = 512
ROWCAP = 321536
CNTP = 384
ACCR = 336

_i32 = jnp.int32


def _iota16():
  return lax.iota(_i32, 16)


def _splat(x):
  return jnp.zeros((16,), _i32) + x


def _append_ring(ref, ocur, vals, mask):
  rank = plsc.cumsum(jnp.ones((16,), _i32), mask=mask)
  pos = (ocur + rank - 1) & (OCAP - 1)
  plsc.store_scatter(ref, [pos], vals, mask=mask)
  return ocur + plsc.all_reduce_population_count(mask)


def _k1_body(col_hbm, src_hbm, dst_hbm, pairs_hbm, lens_hbm, cnt_hbm,
             col_v, bm, pend, ostage, cnt_v, sbuf, dbuf, sbuf2, dbuf2, lenv,
             sem_as, sem_ad, sem_bs, sem_bd):
  core = lax.axis_index("c")
  sub = lax.axis_index("s")
  wid = sub * 2 + core
  lo = wid * CPT
  hi = jnp.minimum(lo + CPT, N)
  iot = _iota16()

  pltpu.sync_copy(col_hbm, col_v)

  def _zero_bm(i, _):
    bm[pl.ds(i * 16, 16)] = jnp.zeros((16,), _i32)
    return 0
  lax.fori_loop(0, BMW // 16, _zero_bm, 0)

  def _zero_cnt(i, _):
    cnt_v[pl.ds(i * 16, 16)] = jnp.zeros((16,), _i32)
    return 0
  lax.fori_loop(0, CNTP // 16, _zero_cnt, 0)

  def _self(i, ocur):
    k = i * 16 + iot
    m = k < (hi - lo)
    kc = jnp.where(m, k, 0)
    c = lo + kc
    word = kc * WPC + (c >> 5)
    bit = jnp.left_shift(_i32(1), c & 31)
    plsc.store_scatter(bm, [word], bit, mask=m)
    vn = plsc.load_gather(col_v, [c], mask=m)
    keep = m & (vn == 0.0)
    plsc.addupdate_scatter(cnt_v, [kc], jnp.ones((16,), _i32), mask=keep)
    return _append_ring(ostage, ocur, (c << 14) | c, mask=keep)
  ocur = lax.fori_loop(0, (CPT + 15) // 16, _self, jnp.zeros((16,), _i32))

  def _consume(pcur, ocur):
    def body(i, oc):
      base = i * 16
      lv = (base + iot) < pcur
      p = pend[pl.ds(base, 16)]
      c = jnp.where(lv, p >> 14, lo)
      n = jnp.where(lv, p & 16383, 0)
      k = c - lo
      word = k * WPC + (n >> 5)
      bit = jnp.left_shift(_i32(1), n & 31)
      rc, _ = plsc.scan_count(p, mask=lv)
      first = lv & (rc == 1)
      w = plsc.load_gather(bm, [word], mask=lv)
      fresh = first & ((w & bit) == 0)
      def wave_cond(a):
        return jnp.max(a) > 0
      def wave_body(a):
        ab = a > 0
        rcw, _ = plsc.scan_count(word, mask=ab)
        lead = ab & (rcw == 1)
        plsc.addupdate_scatter(bm, [word], bit, mask=lead)
        return jnp.where(lead, 0, a)
      lax.while_loop(wave_cond, wave_body, jnp.where(fresh, 1, 0))
      vn = plsc.load_gather(col_v, [n], mask=lv)
      keep = fresh & (vn == 0.0)
      rc2, last2 = plsc.scan_count(k, mask=keep)
      plsc.addupdate_scatter(cnt_v, [k], rc2, mask=keep & last2)
      return _append_ring(ostage, oc, p, mask=keep)
    return lax.fori_loop(0, (pcur + 15) // 16, body, ocur)

  def _process(sbuf, dbuf, carry):
    ocur, flushed = carry

    nc = hi - lo
    @plsc.parallel_loop(0, CHUNK // 16, unroll=4,
                        carry=jnp.zeros((16,), _i32))
    def scan(j, pcur):
      s = sbuf[pl.ds(j * 16, 16)]
      d = dbuf[pl.ds(j * 16, 16)]
      for c, n in ((s, d), (d, s)):
        m = (c - lo).astype(jnp.uint32) < nc.astype(jnp.uint32)
        pk = (c << 14) | n
        rank = plsc.cumsum(jnp.ones((16,), _i32), mask=m)
        plsc.store_scatter(pend, [pcur + rank - 1], pk, mask=m)
        pcur = pcur + plsc.all_reduce_population_count(m)
      return pcur
    ocur = _consume(jnp.max(scan), ocur)

    def fl_cond(st):
      oc, fl = st
      return (oc - fl) >= FLUSH
    def fl_body(st):
      oc, fl = st
      pltpu.sync_copy(ostage.at[pl.ds(pl.multiple_of(fl & (OCAP - 1), 512), FLUSH)],
                      pairs_hbm.at[pl.ds(pl.multiple_of(wid * ROWCAP + fl, 512), FLUSH)])
      return oc, fl + FLUSH
    ocur_s, flushed = lax.while_loop(fl_cond, fl_body, (jnp.max(ocur), flushed))
    return ocur, flushed

  def _start(ch, sb, db, ss, sd):
    off = pl.multiple_of(ch * CHUNK, 8)
    pltpu.async_copy(src_hbm.at[pl.ds(off, CHUNK)], sb, ss)
    pltpu.async_copy(dst_hbm.at[pl.ds(off, CHUNK)], db, sd)

  def _waitc(ch, sb, db, ss, sd):
    off = pl.multiple_of(ch * CHUNK, 8)
    pltpu.make_async_copy(src_hbm.at[pl.ds(off, CHUNK)], sb, ss).wait()
    pltpu.make_async_copy(dst_hbm.at[pl.ds(off, CHUNK)], db, sd).wait()

  _start(0, sbuf, dbuf, sem_as, sem_ad)

  def _ring(i, carry):
    cha = 2 * i
    _waitc(cha, sbuf, dbuf, sem_as, sem_ad)
    _start(cha + 1, sbuf2, dbuf2, sem_bs, sem_bd)
    carry = _process(sbuf, dbuf, carry)
    _waitc(cha + 1, sbuf2, dbuf2, sem_bs, sem_bd)
    _start(jnp.minimum(cha + 2, NCH - 2), sbuf, dbuf, sem_as, sem_ad)
    carry = _process(sbuf2, dbuf2, carry)
    return carry

  ocur, flushed = lax.fori_loop(0, NCH // 2, _ring, (ocur, _i32(0)))
  _waitc(NCH - 2, sbuf, dbuf, sem_as, sem_ad)

  pltpu.sync_copy(ostage.at[pl.ds(pl.multiple_of(flushed & (OCAP - 1), 512), FLUSH)],
                  pairs_hbm.at[pl.ds(pl.multiple_of(wid * ROWCAP + flushed, 512), FLUSH)])

  for j in range(8):
    lenv[pl.ds(j * 16, 16)] = ocur
  pltpu.sync_copy(lenv, lens_hbm.at[pl.ds(pl.multiple_of(wid * 128, 128), 128)])
  pltpu.sync_copy(cnt_v, cnt_hbm.at[pl.ds(pl.multiple_of(wid * CNTP, 128), CNTP)])


def _k2_body(x_hbm, zeros_hbm, pairs_hbm, lens_hbm, cnt_hbm, out_hbm,
             stage, pendb, accall, outall, cntv, lenv, acc_sh):
  core = lax.axis_index("c")
  sub = lax.axis_index("s")
  wid = sub * 2 + core
  lo = wid * CPT
  hi = jnp.minimum(lo + CPT, N)
  cw = hi - lo
  iot = _iota16()
  arow = sub * ACCR

  pltpu.sync_copy(zeros_hbm, acc_sh.at[pl.ds(pl.multiple_of(arow, 16), ACCR)])

  pltpu.sync_copy(lens_hbm.at[pl.ds(pl.multiple_of(wid * 128, 128), 128)], lenv)
  ln = jnp.max(lenv[pl.ds(0, 16)])
  pltpu.sync_copy(cnt_hbm.at[pl.ds(pl.multiple_of(wid * CNTP, 128), CNTP)], cntv)

  def ch_cond(ch):
    return ch * 2048 < ln
  def ch_body(ch):
    pltpu.sync_copy(pairs_hbm.at[pl.ds(pl.multiple_of(wid * ROWCAP + ch * 2048, 512), 2048)], pendb)
    def batch(b, _):
      g0 = ch * 2048 + b * 16
      gm = (g0 + iot) < ln
      p = pendb[pl.ds(b * 16, 16)]
      n = jnp.where(gm, p & 16383, 0)
      k = jnp.where(gm, (p >> 14) - lo, CPT)
      pltpu.sync_copy(x_hbm.at[n], stage)
      pltpu.sync_copy(stage, acc_sh.at[arow + k], add=True)
      return 0
    lax.fori_loop(0, 128, batch, 0)
    return ch + 1
  lax.while_loop(ch_cond, ch_body, _i32(0))

  pltpu.sync_copy(acc_sh.at[pl.ds(pl.multiple_of(arow, 16), CPT)], accall)

  @plsc.parallel_loop(0, CPT, unroll=2)
  def _row(r):
    cs = plsc.load_gather(cntv, [_splat(r)])
    cf = cs.astype(jnp.float32)
    for j in range(D // 16):
      v = accall[r, pl.ds(j * 16, 16)]
      col = j * 16 + iot
      plsc.store_scatter(outall, [_splat(r), col], v / cf,
                         mask=col < DO)

  for t in range(CPT // 80):
    row0 = jnp.minimum(t * 80, cw - 80)
    pltpu.sync_copy(outall.at[pl.ds(pl.multiple_of(row0, 8), 80)],
                    out_hbm.at[pl.ds(pl.multiple_of(lo + row0, 8), 80)])


def _mesh():
  return plsc.VectorSubcoreMesh(core_axis_name="c", subcore_axis_name="s")


_CP = pltpu.CompilerParams(use_tc_tiling_on_sc=False,
                           needs_layout_passes=False)


@jax.jit
def kernel(x, edge_index):
  col = x[:, D - 1]
  k1 = pl.kernel(
      _k1_body,
      out_type=(
          jax.ShapeDtypeStruct((NT * ROWCAP,), _i32),
          jax.ShapeDtypeStruct((NT * 128,), _i32),
          jax.ShapeDtypeStruct((NT * CNTP,), _i32),
      ),
      mesh=_mesh(),
      compiler_params=_CP,
      scratch_types=[
          pltpu.VMEM((N,), jnp.float32),
          pltpu.VMEM((BMW,), _i32),
          pltpu.VMEM((PCAP,), _i32),
          pltpu.VMEM((OCAP,), _i32),
          pltpu.VMEM((CNTP,), _i32),
          pltpu.VMEM((CHUNK,), _i32),
          pltpu.VMEM((CHUNK,), _i32),
          pltpu.VMEM((CHUNK,), _i32),
          pltpu.VMEM((CHUNK,), _i32),
          pltpu.VMEM((128,), _i32),
          pltpu.SemaphoreType.DMA,
          pltpu.SemaphoreType.DMA,
          pltpu.SemaphoreType.DMA,
          pltpu.SemaphoreType.DMA,
      ],
  )
  pairs, lens, cnt = k1(col, edge_index[0], edge_index[1])

  k2 = pl.kernel(
      _k2_body,
      out_type=jax.ShapeDtypeStruct((N, DO), jnp.float32),
      mesh=_mesh(),
      compiler_params=_CP,
      scratch_types=[
          pltpu.VMEM((16, D), jnp.float32),
          pltpu.VMEM((2048,), _i32),
          pltpu.VMEM((CPT, D), jnp.float32),
          pltpu.VMEM((CPT, DO), jnp.float32),
          pltpu.VMEM((CNTP,), _i32),
          pltpu.VMEM((128,), _i32),
          pltpu.VMEM_SHARED((16 * ACCR, D), jnp.float32),
      ],
  )
  zeros = jnp.zeros((ACCR, D), jnp.float32)
  return k2(x, zeros, pairs, lens, cnt)


# speedup vs baseline: 1.3388x; 1.1183x over previous
import jax
import jax.numpy as jnp
from jax import lax
from jax.experimental import pallas as pl
from jax.experimental.pallas import tpu as pltpu
from jax.experimental.pallas import tpu_sc as plsc

N = 10000
E = 160000
D = 128
DO = 127

NT = 32
CPT = 320
WPC = 313
BMW = 100160
CHUNK = 2000
NCH = E // CHUNK
PCAP = 4032
OCAP = 4096
FLUSH = 512
ROWCAP = 321536
CNTP = 384
ACCR = 336

_i32 = jnp.int32


def _iota16():
  return lax.iota(_i32, 16)


def _splat(x):
  return jnp.zeros((16,), _i32) + x


def _append_ring(ref, ocur, vals, mask):
  rank = plsc.cumsum(jnp.ones((16,), _i32), mask=mask)
  pos = (ocur + rank - 1) & (OCAP - 1)
  plsc.store_scatter(ref, [pos], vals, mask=mask)
  return ocur + plsc.all_reduce_population_count(mask)


def _k1_body(col_hbm, src_hbm, dst_hbm, zbm_hbm, pairs_hbm, lens_hbm, cnt_hbm,
             col_v, bm, pend, ostage, cnt_v, sbuf, dbuf, sbuf2, dbuf2, lenv,
             sem_as, sem_ad, sem_bs, sem_bd):
  core = lax.axis_index("c")
  sub = lax.axis_index("s")
  wid = sub * 2 + core
  lo = wid * CPT
  hi = jnp.minimum(lo + CPT, N)
  iot = _iota16()

  pltpu.sync_copy(col_hbm, col_v)

  pltpu.sync_copy(zbm_hbm, bm)

  def _zero_cnt(i, _):
    cnt_v[pl.ds(i * 16, 16)] = jnp.zeros((16,), _i32)
    return 0
  lax.fori_loop(0, CNTP // 16, _zero_cnt, 0)

  def _self(i, ocur):
    k = i * 16 + iot
    m = k < (hi - lo)
    kc = jnp.where(m, k, 0)
    c = lo + kc
    word = kc * WPC + (c >> 5)
    bit = jnp.left_shift(_i32(1), c & 31)
    plsc.store_scatter(bm, [word], bit, mask=m)
    vn = plsc.load_gather(col_v, [c], mask=m)
    keep = m & (vn == 0.0)
    plsc.addupdate_scatter(cnt_v, [kc], jnp.ones((16,), _i32), mask=keep)
    return _append_ring(ostage, ocur, (c << 14) | c, mask=keep)
  ocur = lax.fori_loop(0, (CPT + 15) // 16, _self, jnp.zeros((16,), _i32))

  def _consume(pcur, ocur):
    def body(i, oc):
      base = i * 16
      lv = (base + iot) < pcur
      p = pend[pl.ds(base, 16)]
      c = jnp.where(lv, p >> 14, lo)
      n = jnp.where(lv, p & 16383, 0)
      k = c - lo
      word = k * WPC + (n >> 5)
      bit = jnp.left_shift(_i32(1), n & 31)
      rc, _ = plsc.scan_count(p, mask=lv)
      first = lv & (rc == 1)
      w = plsc.load_gather(bm, [word], mask=lv)
      fresh = first & ((w & bit) == 0)
      def wave_cond(a):
        return jnp.max(a) > 0
      def wave_body(a):
        ab = a > 0
        rcw, _ = plsc.scan_count(word, mask=ab)
        lead = ab & (rcw == 1)
        plsc.addupdate_scatter(bm, [word], bit, mask=lead)
        return jnp.where(lead, 0, a)
      lax.while_loop(wave_cond, wave_body, jnp.where(fresh, 1, 0))
      vn = plsc.load_gather(col_v, [n], mask=lv)
      keep = fresh & (vn == 0.0)
      rc2, last2 = plsc.scan_count(k, mask=keep)
      plsc.addupdate_scatter(cnt_v, [k], rc2, mask=keep & last2)
      return _append_ring(ostage, oc, p, mask=keep)
    return lax.fori_loop(0, (pcur + 15) // 16, body, ocur)

  def _process(sbuf, dbuf, carry):
    ocur, flushed = carry

    nc = hi - lo
    @plsc.parallel_loop(0, CHUNK // 16, unroll=4,
                        carry=jnp.zeros((16,), _i32))
    def scan(j, pcur):
      s = sbuf[pl.ds(j * 16, 16)]
      d = dbuf[pl.ds(j * 16, 16)]
      for c, n in ((s, d), (d, s)):
        m = (c - lo).astype(jnp.uint32) < nc.astype(jnp.uint32)
        pk = (c << 14) | n
        rank = plsc.cumsum(jnp.ones((16,), _i32), mask=m)
        plsc.store_scatter(pend, [pcur + rank - 1], pk, mask=m)
        pcur = pcur + plsc.all_reduce_population_count(m)
      return pcur
    ocur = _consume(jnp.max(scan), ocur)

    def fl_cond(st):
      oc, fl = st
      return (oc - fl) >= FLUSH
    def fl_body(st):
      oc, fl = st
      pltpu.sync_copy(ostage.at[pl.ds(pl.multiple_of(fl & (OCAP - 1), 512), FLUSH)],
                      pairs_hbm.at[pl.ds(pl.multiple_of(wid * ROWCAP + fl, 512), FLUSH)])
      return oc, fl + FLUSH
    ocur_s, flushed = lax.while_loop(fl_cond, fl_body, (jnp.max(ocur), flushed))
    return ocur, flushed

  def _start(ch, sb, db, ss, sd):
    off = pl.multiple_of(ch * CHUNK, 8)
    pltpu.async_copy(src_hbm.at[pl.ds(off, CHUNK)], sb, ss)
    pltpu.async_copy(dst_hbm.at[pl.ds(off, CHUNK)], db, sd)

  def _waitc(ch, sb, db, ss, sd):
    off = pl.multiple_of(ch * CHUNK, 8)
    pltpu.make_async_copy(src_hbm.at[pl.ds(off, CHUNK)], sb, ss).wait()
    pltpu.make_async_copy(dst_hbm.at[pl.ds(off, CHUNK)], db, sd).wait()

  _start(0, sbuf, dbuf, sem_as, sem_ad)

  def _ring(i, carry):
    cha = 2 * i
    _waitc(cha, sbuf, dbuf, sem_as, sem_ad)
    _start(cha + 1, sbuf2, dbuf2, sem_bs, sem_bd)
    carry = _process(sbuf, dbuf, carry)
    _waitc(cha + 1, sbuf2, dbuf2, sem_bs, sem_bd)
    _start(jnp.minimum(cha + 2, NCH - 2), sbuf, dbuf, sem_as, sem_ad)
    carry = _process(sbuf2, dbuf2, carry)
    return carry

  ocur, flushed = lax.fori_loop(0, NCH // 2, _ring, (ocur, _i32(0)))
  _waitc(NCH - 2, sbuf, dbuf, sem_as, sem_ad)

  pltpu.sync_copy(ostage.at[pl.ds(pl.multiple_of(flushed & (OCAP - 1), 512), FLUSH)],
                  pairs_hbm.at[pl.ds(pl.multiple_of(wid * ROWCAP + flushed, 512), FLUSH)])

  for j in range(8):
    lenv[pl.ds(j * 16, 16)] = ocur
  pltpu.sync_copy(lenv, lens_hbm.at[pl.ds(pl.multiple_of(wid * 128, 128), 128)])
  pltpu.sync_copy(cnt_v, cnt_hbm.at[pl.ds(pl.multiple_of(wid * CNTP, 128), CNTP)])


def _k2_body(x_hbm, zeros_hbm, pairs_hbm, lens_hbm, cnt_hbm, out_hbm,
             stage, pendb, accall, outall, cntv, lenv, acc_sh):
  core = lax.axis_index("c")
  sub = lax.axis_index("s")
  wid = sub * 2 + core
  lo = wid * CPT
  hi = jnp.minimum(lo + CPT, N)
  cw = hi - lo
  iot = _iota16()
  arow = sub * ACCR

  pltpu.sync_copy(zeros_hbm, acc_sh.at[pl.ds(pl.multiple_of(arow, 16), ACCR)])

  pltpu.sync_copy(lens_hbm.at[pl.ds(pl.multiple_of(wid * 128, 128), 128)], lenv)
  ln = jnp.max(lenv[pl.ds(0, 16)])
  pltpu.sync_copy(cnt_hbm.at[pl.ds(pl.multiple_of(wid * CNTP, 128), CNTP)], cntv)

  def ch_cond(ch):
    return ch * 2048 < ln
  def ch_body(ch):
    pltpu.sync_copy(pairs_hbm.at[pl.ds(pl.multiple_of(wid * ROWCAP + ch * 2048, 512), 2048)], pendb)
    def batch(b, _):
      g0 = ch * 2048 + b * 16
      gm = (g0 + iot) < ln
      p = pendb[pl.ds(b * 16, 16)]
      n = jnp.where(gm, p & 16383, 0)
      k = jnp.where(gm, (p >> 14) - lo, CPT)
      pltpu.sync_copy(x_hbm.at[n], stage)
      pltpu.sync_copy(stage, acc_sh.at[arow + k], add=True)
      return 0
    lax.fori_loop(0, 128, batch, 0)
    return ch + 1
  lax.while_loop(ch_cond, ch_body, _i32(0))

  pltpu.sync_copy(acc_sh.at[pl.ds(pl.multiple_of(arow, 16), CPT)], accall)

  @plsc.parallel_loop(0, CPT, unroll=2)
  def _row(r):
    cs = plsc.load_gather(cntv, [_splat(r)])
    cf = cs.astype(jnp.float32)
    for j in range(D // 16):
      v = accall[r, pl.ds(j * 16, 16)]
      col = j * 16 + iot
      plsc.store_scatter(outall, [_splat(r), col], v / cf,
                         mask=col < DO)

  for t in range(CPT // 80):
    row0 = jnp.minimum(t * 80, cw - 80)
    pltpu.sync_copy(outall.at[pl.ds(pl.multiple_of(row0, 8), 80)],
                    out_hbm.at[pl.ds(pl.multiple_of(lo + row0, 8), 80)])


def _mesh():
  return plsc.VectorSubcoreMesh(core_axis_name="c", subcore_axis_name="s")


_CP = pltpu.CompilerParams(use_tc_tiling_on_sc=False,
                           needs_layout_passes=False)


@jax.jit
def kernel(x, edge_index):
  col = x[:, D - 1]
  k1 = pl.kernel(
      _k1_body,
      out_type=(
          jax.ShapeDtypeStruct((NT * ROWCAP,), _i32),
          jax.ShapeDtypeStruct((NT * 128,), _i32),
          jax.ShapeDtypeStruct((NT * CNTP,), _i32),
      ),
      mesh=_mesh(),
      compiler_params=_CP,
      scratch_types=[
          pltpu.VMEM((N,), jnp.float32),
          pltpu.VMEM((BMW,), _i32),
          pltpu.VMEM((PCAP,), _i32),
          pltpu.VMEM((OCAP,), _i32),
          pltpu.VMEM((CNTP,), _i32),
          pltpu.VMEM((CHUNK,), _i32),
          pltpu.VMEM((CHUNK,), _i32),
          pltpu.VMEM((CHUNK,), _i32),
          pltpu.VMEM((CHUNK,), _i32),
          pltpu.VMEM((128,), _i32),
          pltpu.SemaphoreType.DMA,
          pltpu.SemaphoreType.DMA,
          pltpu.SemaphoreType.DMA,
          pltpu.SemaphoreType.DMA,
      ],
  )
  zbm = jnp.zeros((BMW,), _i32)
  pairs, lens, cnt = k1(col, edge_index[0], edge_index[1], zbm)

  k2 = pl.kernel(
      _k2_body,
      out_type=jax.ShapeDtypeStruct((N, DO), jnp.float32),
      mesh=_mesh(),
      compiler_params=_CP,
      scratch_types=[
          pltpu.VMEM((16, D), jnp.float32),
          pltpu.VMEM((2048,), _i32),
          pltpu.VMEM((CPT, D), jnp.float32),
          pltpu.VMEM((CPT, DO), jnp.float32),
          pltpu.VMEM((CNTP,), _i32),
          pltpu.VMEM((128,), _i32),
          pltpu.VMEM_SHARED((16 * ACCR, D), jnp.float32),
      ],
  )
  zeros = jnp.zeros((ACCR, D), jnp.float32)
  return k2(x, zeros, pairs, lens, cnt)
